# baseline (speedup 1.0000x reference)
import functools

import jax
import jax.numpy as jnp
from jax import lax
from jax.experimental import pallas as pl
from jax.experimental.pallas import tpu as pltpu
from jax.experimental.pallas import tpu_sc as plsc

_N = 10000
_E = 320000
_D = 256
_H = 4
_O = 64
_EPS = 1e-5
_NP = 10240
_BR = 512

_NW = 32
_EP = 327680
_CE = 64
_EC = _EP // _NW
_NCHD = _EC // _CE
_CEA = 256
_CED = 64
_NCHA = (_EP // _NW) // _CEA
_CEE = 32
_ETE = _EP // 16
_NCHE = _ETE // _CEE
_DFR = _NP * 4 // 128
_HN = _NP // 2
_AGR = _HN + 128

_mesh = plsc.VectorSubcoreMesh(core_axis_name="c", subcore_axis_name="s")
_f32 = jnp.float32
_i32 = jnp.int32


def _proj_in_body(x_ref, iss_ref, wx_ref, c_ref, o_ref):
    acc = jnp.dot(x_ref[...], wx_ref[...], preferred_element_type=_f32)
    o_ref[...] = jnp.maximum(acc + iss_ref[...] + c_ref[...], 0.0)


def _proj_in(xp, issrows, wx, c):
    return pl.pallas_call(
        _proj_in_body,
        grid=(_NP // _BR,),
        in_specs=[
            pl.BlockSpec((_BR, 128), lambda i: (i, 0)),
            pl.BlockSpec((_BR, _D), lambda i: (i, 0)),
            pl.BlockSpec((128, _D), lambda i: (0, 0)),
            pl.BlockSpec((1, _D), lambda i: (0, 0)),
        ],
        out_specs=pl.BlockSpec((_BR, _D), lambda i: (i, 0)),
        out_shape=jax.ShapeDtypeStruct((_NP, _D), _f32),
    )(xp, issrows, wx, c)


def _lr_body(h_ref, wl_ref, wr_ref, bl_ref, br_ref, xl_ref, xr_ref):
    h = h_ref[...]
    xl_ref[...] = jnp.dot(h, wl_ref[...], preferred_element_type=_f32) + bl_ref[...]
    xr_ref[...] = jnp.dot(h, wr_ref[...], preferred_element_type=_f32) + br_ref[...]


def _proj_lr(h, wl, wr, bl, br):
    return pl.pallas_call(
        _lr_body,
        grid=(_NP // _BR,),
        in_specs=[
            pl.BlockSpec((_BR, _D), lambda i: (i, 0)),
            pl.BlockSpec((_D, _D), lambda i: (0, 0)),
            pl.BlockSpec((_D, _D), lambda i: (0, 0)),
            pl.BlockSpec((1, _D), lambda i: (0, 0)),
            pl.BlockSpec((1, _D), lambda i: (0, 0)),
        ],
        out_specs=[
            pl.BlockSpec((_BR, _D), lambda i: (i, 0)),
            pl.BlockSpec((_BR, _D), lambda i: (i, 0)),
        ],
        out_shape=[
            jax.ShapeDtypeStruct((_NP, _D), _f32),
            jax.ShapeDtypeStruct((_NP, _D), _f32),
        ],
    )(h, wl, wr, bl, br)


def _affine_relu_body(a_ref, s_ref, c_ref, o_ref):
    o_ref[...] = jnp.maximum(a_ref[...] * s_ref[...] + c_ref[...], 0.0)


def _affine_relu(agg, s, c):
    return pl.pallas_call(
        _affine_relu_body,
        grid=(_NP // _BR,),
        in_specs=[
            pl.BlockSpec((_BR, _D), lambda i: (i, 0)),
            pl.BlockSpec((1, _D), lambda i: (0, 0)),
            pl.BlockSpec((1, _D), lambda i: (0, 0)),
        ],
        out_specs=pl.BlockSpec((_BR, _D), lambda i: (i, 0)),
        out_shape=jax.ShapeDtypeStruct((_NP, _D), _f32),
    )(agg, s, c)


@functools.partial(
    pl.kernel,
    out_type=jax.ShapeDtypeStruct((_NP, _D), _f32),
    mesh=_mesh,
    compiler_params=pltpu.CompilerParams(needs_layout_passes=False),
    scratch_types=[
        pltpu.VMEM((_NP // _NW,), _i32),
        pltpu.VMEM((_NP // _NW, _D), _f32),
        pltpu.SemaphoreType.DMA,
    ],
)
def _iss_gather(emb_hbm, idx_hbm, out_hbm, idx_v, rows_v, sem):
    wid = lax.axis_index("s") * 2 + lax.axis_index("c")
    base = wid * (_NP // _NW)
    pltpu.sync_copy(idx_hbm.at[pl.ds(base, _NP // _NW)], idx_v)
    pltpu.async_copy(emb_hbm.at[idx_v], rows_v, sem).wait()
    pltpu.sync_copy(rows_v, out_hbm.at[pl.ds(base, _NP // _NW)])


def _edge_logits_body(xl_hbm, xr_hbm, ep4_hbm, r8_hbm, u_hbm, att_hbm,
                      g16_hbm, ex_hbm,
                      src0_v, src1_v, src2_v, dst0_v, dst1_v, dst2_v,
                      ep4f_v, exf_v, xl0_v, xl1_v, xl2_v, xr0_v, xr1_v, xr2_v,
                      r8_v, u_v, att_v, g16_v,
                      gl0_sem, gl1_sem, gl2_sem, gr0_sem, gr1_sem, gr2_sem):
    c = lax.axis_index("c")
    s = lax.axis_index("s")
    wid = s * 2 + c
    pltpu.sync_copy(r8_hbm, r8_v)
    pltpu.sync_copy(u_hbm, u_v)
    pltpu.sync_copy(att_hbm, att_v)
    pltpu.sync_copy(g16_hbm, g16_v)

    iota = lax.iota(_i32, 16)
    zid = jnp.zeros((16,), _i32)
    srcs = (src0_v, src1_v, src2_v)
    dsts = (dst0_v, dst1_v, dst2_v)
    xls = (xl0_v, xl1_v, xl2_v)
    xrs = (xr0_v, xr1_v, xr2_v)
    glsems = (gl0_sem, gl1_sem, gl2_sem)
    grsems = (gr0_sem, gr1_sem, gr2_sem)
    nchunk = _EC // _CE
    ngrp = nchunk // 4
    erow4 = _EC * 4 // 128

    def extract(k):
        b = k % 3
        for g in range(_CE // 16):
            sl = pl.ds(g * 16, 16)
            f4 = (k * _CE + g * 16 + iota) * 4
            sv = plsc.load_gather(ep4f_v, [f4 >> 7, f4 & 127])
            dv = plsc.load_gather(ep4f_v, [f4 >> 7, (f4 & 127) + 1])
            srcs[b][sl] = sv
            dsts[b][sl] = dv

    def launch(k):
        b = k % 3
        return (pltpu.async_copy(xl_hbm.at[srcs[b]], xls[b], glsems[b]),
                pltpu.async_copy(xr_hbm.at[dsts[b]], xrs[b], grsems[b]))

    def grp(og, _):
        pltpu.sync_copy(
            ep4_hbm.at[pl.ds(wid * erow4 + og * (_CE * 4 * 4 // 128),
                             _CE * 4 * 4 // 128)], ep4f_v)
        pend_g = {}
        extract(0)
        pend_g[0] = launch(0)
        extract(1)
        pend_g[1] = launch(1)
        for k in range(4):
            b = k % 3
            if k < 2:
                extract(k + 2)
                pend_g[(k + 2) % 3] = launch(k + 2)
            pend_g[b][0].wait()
            pend_g[b][1].wait()
            tgs, wgs = [], []
            for g in range(_CE // 16):
                f4 = (k * _CE + g * 16 + iota) * 4
                tv = plsc.load_gather(ep4f_v, [f4 >> 7, (f4 & 127) + 2])
                wv = plsc.bitcast(
                    plsc.load_gather(ep4f_v, [f4 >> 7, (f4 & 127) + 3]), _f32)
                gg = plsc.load_gather(g16_v, [zid, tv])
                tgs.append(tv)
                wgs.append(wv * gg)
            for h in range(_H):

                @plsc.parallel_loop(0, _O, 1, unroll=4,
                                    carry=tuple(jnp.zeros((16,), _f32)
                                                for _ in range(_CE // 16)))
                def accs(j, acc, b=b, h=h, tgs=tgs, wgs=wgs):
                    colv = jnp.full((16,), h * _O, _i32) + j
                    uj = plsc.load_gather(u_v, [zid, colv])
                    aj = plsc.load_gather(att_v, [zid, colv])
                    out = []
                    for g in range(_CE // 16):
                        iog = iota + g * 16
                        xlg = plsc.load_gather(xls[b], [iog, colv])
                        xrg = plsc.load_gather(xrs[b], [iog, colv])
                        r8g = plsc.load_gather(r8_v, [tgs[g], colv])
                        m = xlg + xrg + r8g + wgs[g] * uj
                        m = jnp.where(m >= 0.0, m, m * 0.2)
                        out.append(acc[g] + aj * m)
                    return tuple(out)

                for g in range(_CE // 16):
                    exv = jnp.exp(accs[g])
                    f = (k * _CE + g * 16 + iota) * 4 + h
                    plsc.store_scatter(exf_v, [f >> 7, f & 127], exv)
        pltpu.sync_copy(
            exf_v,
            ex_hbm.at[pl.ds(wid * erow4 + og * (_CE * 4 * 4 // 128),
                            _CE * 4 * 4 // 128)])
        return 0

    lax.fori_loop(0, ngrp, grp, 0)


_edge_logits = functools.partial(
    pl.kernel,
    out_type=jax.ShapeDtypeStruct((_EP * 4 // 128, 128), _f32),
    mesh=_mesh,
    compiler_params=pltpu.CompilerParams(needs_layout_passes=False),
    scratch_types=[
        pltpu.VMEM((_CE,), _i32),
        pltpu.VMEM((_CE,), _i32),
        pltpu.VMEM((_CE,), _i32),
        pltpu.VMEM((_CE,), _i32),
        pltpu.VMEM((_CE,), _i32),
        pltpu.VMEM((_CE,), _i32),
        pltpu.VMEM((_CE * 4 * 4 // 128, 128), _i32),
        pltpu.VMEM((_CE * 4 * 4 // 128, 128), _f32),
        pltpu.VMEM((_CE, _D), _f32),
        pltpu.VMEM((_CE, _D), _f32),
        pltpu.VMEM((_CE, _D), _f32),
        pltpu.VMEM((_CE, _D), _f32),
        pltpu.VMEM((_CE, _D), _f32),
        pltpu.VMEM((_CE, _D), _f32),
        pltpu.VMEM((8, _D), _f32),
        pltpu.VMEM((1, _D), _f32),
        pltpu.VMEM((1, _D), _f32),
        pltpu.VMEM((1, 128), _f32),
        pltpu.SemaphoreType.DMA,
        pltpu.SemaphoreType.DMA,
        pltpu.SemaphoreType.DMA,
        pltpu.SemaphoreType.DMA,
        pltpu.SemaphoreType.DMA,
        pltpu.SemaphoreType.DMA,
    ],
)(_edge_logits_body)


def _den_body(ex_hbm, ep2_hbm, z128_hbm, den01_hbm,
              dst0_v, dst1_v, dst2_v, dst3_v, ep2f_v, exf_v,
              exa_v, exb_v, den_sh, sd0_sem, sd1_sem):
    c = lax.axis_index("c")
    s = lax.axis_index("s")
    zsl = pl.ds(s * (_NP // 16), _NP // 16)
    pltpu.sync_copy(z128_hbm.at[zsl], den_sh.at[zsl])
    pltpu.sync_copy(z128_hbm.at[pl.ds(0, _CED)], exa_v)
    pltpu.sync_copy(z128_hbm.at[pl.ds(0, _CED)], exb_v)
    plsc.subcore_barrier()
    wid = s * 2 + c

    iota = lax.iota(_i32, 16)
    dsts = (dst0_v, dst1_v, dst2_v, dst3_v)
    exs = (exa_v, exb_v)
    sdsems = (sd0_sem, sd1_sem)
    epw = _EC * 2 // 128
    exw = _EC * 4 // 128

    def grp(og, _):
        pltpu.sync_copy(
            ep2_hbm.at[pl.ds(wid * epw + og * (_CED * 8 * 2 // 128),
                             _CED * 8 * 2 // 128)], ep2f_v)
        pltpu.sync_copy(
            ex_hbm.at[pl.ds(wid * exw + og * (_CED * 8 * 4 // 128),
                            _CED * 8 * 4 // 128)], exf_v)
        pend = {0: None, 1: None}
        for k in range(8):
            b = k % 2
            b4 = k % 4
            for g in range(_CED // 16):
                sl = pl.ds(g * 16, 16)
                f2 = (k * _CED + g * 16 + iota) * 2
                dv = plsc.load_gather(ep2f_v, [f2 >> 7, (f2 & 127) + 1])
                dsts[b4][sl] = dv
            if pend[b] is not None:
                pend[b].wait()
            for v in range(_CED * 4 // 16):
                f = v * 16 + k * _CED * 4 + iota
                fl = v * 16 + iota
                exv = plsc.load_gather(exf_v, [f >> 7, f & 127])
                plsc.store_scatter(exs[b], [fl >> 2, fl & 3], exv)
            pend[b] = pltpu.async_copy(exs[b], den_sh.at[dsts[b4]],
                                       sdsems[b], add=True)
        for b in (0, 1):
            if pend[b] is not None:
                pend[b].wait()
        return 0

    lax.fori_loop(0, _EC // (_CED * 8), grp, 0)
    plsc.subcore_barrier()
    pltpu.sync_copy(den_sh.at[zsl],
                    den01_hbm.at[pl.ds(c * _NP + s * (_NP // 16), _NP // 16)])


_den_acc = functools.partial(
    pl.kernel,
    out_type=jax.ShapeDtypeStruct((2 * _NP, 128), _f32),
    mesh=_mesh,
    compiler_params=pltpu.CompilerParams(needs_layout_passes=False),
    scratch_types=[
        pltpu.VMEM((_CED,), _i32),
        pltpu.VMEM((_CED,), _i32),
        pltpu.VMEM((_CED,), _i32),
        pltpu.VMEM((_CED,), _i32),
        pltpu.VMEM((_CED * 8 * 2 // 128, 128), _i32),
        pltpu.VMEM((_CED * 8 * 4 // 128, 128), _f32),
        pltpu.VMEM((_CED, 128), _f32),
        pltpu.VMEM((_CED, 128), _f32),
        pltpu.VMEM_SHARED((_NP, 128), _f32),
        pltpu.SemaphoreType.DMA,
        pltpu.SemaphoreType.DMA,
    ],
)(_den_body)


def _alpha_body(ex_hbm, dst_hbm, den4f_hbm, af_hbm,
                exf_v, dst_v, den4f_v, af_v):
    c = lax.axis_index("c")
    s = lax.axis_index("s")
    wid = s * 2 + c
    pltpu.sync_copy(den4f_hbm, den4f_v)
    iota = lax.iota(_i32, 16)

    def chunk(ci, _):
        base = wid * (_EP // _NW) + ci * _CEA
        rowb = wid * (_EP // _NW * 4 // 128) + ci * (_CEA * 4 // 128)
        pltpu.sync_copy(dst_hbm.at[pl.ds(base, _CEA)], dst_v)
        pltpu.sync_copy(ex_hbm.at[pl.ds(rowb, _CEA * 4 // 128)], exf_v)
        for v in range(_CEA * 4 // 16):
            f = iota + v * 16
            e = f >> 2
            hh = f & 3
            exv = plsc.load_gather(exf_v, [f >> 7, f & 127])
            dg = plsc.load_gather(dst_v, [e])
            gf = dg * 4 + hh
            den = plsc.load_gather(den4f_v, [gf >> 7, gf & 127])
            plsc.store_scatter(af_v, [f >> 7, f & 127], exv / den)
        pltpu.sync_copy(af_v, af_hbm.at[pl.ds(rowb, _CEA * 4 // 128)])
        return 0

    lax.fori_loop(0, _NCHA, chunk, 0)


_alpha = functools.partial(
    pl.kernel,
    out_type=jax.ShapeDtypeStruct((_EP * 4 // 128, 128), _f32),
    mesh=_mesh,
    compiler_params=pltpu.CompilerParams(needs_layout_passes=False),
    scratch_types=[
        pltpu.VMEM((_CEA * 4 // 128, 128), _f32),
        pltpu.VMEM((_CEA,), _i32),
        pltpu.VMEM((_DFR, 128), _f32),
        pltpu.VMEM((_CEA * 4 // 128, 128), _f32),
    ],
)(_alpha_body)


def _edge_agg_body(xl_hbm, ep2_hbm, af_hbm, zagg_hbm, agg_hbm,
                   src0_v, src1_v, src2_v, la0_v, la1_v, la2_v, la3_v,
                   lb0_v, lb1_v, lb2_v, lb3_v,
                   epf_v, af_v, xl0_v, xl1_v, xl2_v,
                   ma0_v, mb0_v, ma1_v, mb1_v, agg_sh,
                   g0_sem, g1_sem, g2_sem,
                   s0a_sem, s0b_sem, s1a_sem, s1b_sem):
    c = lax.axis_index("c")
    s = lax.axis_index("s")
    lo = c * _HN
    zsl = pl.ds(s * (2 * _AGR // 16), 2 * _AGR // 16)
    pltpu.sync_copy(zagg_hbm.at[zsl], agg_sh.at[zsl])
    plsc.subcore_barrier()

    iota = lax.iota(_i32, 16)
    srcs = (src0_v, src1_v, src2_v)
    las = (la0_v, la1_v, la2_v, la3_v)
    lbs = (lb0_v, lb1_v, lb2_v, lb3_v)
    xls = (xl0_v, xl1_v, xl2_v)
    mas = (ma0_v, ma1_v)
    mbs = (mb0_v, mb1_v)
    gsems = (g0_sem, g1_sem, g2_sem)
    sasems = (s0a_sem, s0b_sem)
    sbsems = (s1a_sem, s1b_sem)
    ngrp = _NCHE // 16
    erow2 = _ETE * 2 // 128
    erow4 = _ETE * 4 // 128

    def extract(k):
        b = k % 3
        b4 = k % 4
        for g in range(_CEE // 16):
            sl = pl.ds(g * 16, 16)
            le = k * _CEE + g * 16 + iota
            f2 = le * 2
            sv = plsc.load_gather(epf_v, [f2 >> 7, f2 & 127])
            dv = plsc.load_gather(epf_v, [f2 >> 7, (f2 & 127) + 1])
            srcs[b][sl] = sv
            inr = (dv >= lo) & (dv < lo + _HN)
            loc2 = jnp.where(inr, dv - lo, _HN) * 2
            las[b4][sl] = loc2
            lbs[b4][sl] = loc2 + 1

    def launch(k):
        b = k % 3
        return pltpu.async_copy(xl_hbm.at[srcs[b]], xls[b], gsems[b])

    def grp(og, _):
        pltpu.sync_copy(
            ep2_hbm.at[pl.ds(s * erow2 + og * 8, 8)], epf_v)
        pltpu.sync_copy(
            af_hbm.at[pl.ds(s * erow4 + og * 16, 16)], af_v)
        pend_g = {}
        pend_sa = {0: None, 1: None}
        extract(0)
        pend_g[0] = launch(0)
        extract(1)
        pend_g[1] = launch(1)
        for k in range(16):
            b = k % 2
            g3 = k % 3
            if k < 14:
                extract(k + 2)
                pend_g[(k + 2) % 3] = launch(k + 2)
            pend_g[g3].wait()
            alphas = []
            for h in range(_H):
                row = []
                for g in range(_CEE // 16):
                    f = (k * _CEE + g * 16 + iota) * 4 + h
                    row.append(plsc.load_gather(af_v, [f >> 7, f & 127]))
                alphas.append(row)
            if pend_sa[b] is not None:
                pend_sa[b][0].wait()
                pend_sa[b][1].wait()

            @plsc.parallel_loop(0, _O, 1, unroll=8)
            def _(j, b=b, g3=g3, alphas=alphas):
                mcolv = jnp.full((16,), j, _i32)
                mcolv2 = mcolv + _O
                for h in range(_H):
                    mref = mas[b] if h < 2 else mbs[b]
                    mc = mcolv if h % 2 == 0 else mcolv2
                    colv = mcolv + h * _O
                    for g in range(_CEE // 16):
                        iog = iota + g * 16
                        xlg = plsc.load_gather(xls[g3], [iog, colv])
                        plsc.store_scatter(mref, [iog, mc],
                                           alphas[h][g] * xlg)
            da = pltpu.async_copy(mas[b], agg_sh.at[las[k % 4]], sasems[b],
                                  add=True)
            db = pltpu.async_copy(mbs[b], agg_sh.at[lbs[k % 4]], sbsems[b],
                                  add=True)
            pend_sa[b] = (da, db)
        for b in (0, 1):
            if pend_sa[b] is not None:
                pend_sa[b][0].wait()
                pend_sa[b][1].wait()
        return 0

    lax.fori_loop(0, ngrp, grp, 0)
    plsc.subcore_barrier()
    osl = pl.ds(s * (2 * _HN // 16), 2 * _HN // 16)
    pltpu.sync_copy(
        agg_sh.at[osl],
        agg_hbm.at[pl.ds(2 * lo + s * (2 * _HN // 16), 2 * _HN // 16)])


_edge_agg = functools.partial(
    pl.kernel,
    out_type=jax.ShapeDtypeStruct((2 * _NP, 128), _f32),
    mesh=_mesh,
    compiler_params=pltpu.CompilerParams(needs_layout_passes=False),
    scratch_types=[
        pltpu.VMEM((_CEE,), _i32),
        pltpu.VMEM((_CEE,), _i32),
        pltpu.VMEM((_CEE,), _i32),
        pltpu.VMEM((_CEE,), _i32),
        pltpu.VMEM((_CEE,), _i32),
        pltpu.VMEM((_CEE,), _i32),
        pltpu.VMEM((_CEE,), _i32),
        pltpu.VMEM((_CEE,), _i32),
        pltpu.VMEM((_CEE,), _i32),
        pltpu.VMEM((_CEE,), _i32),
        pltpu.VMEM((_CEE,), _i32),
        pltpu.VMEM((8, 128), _i32),
        pltpu.VMEM((16, 128), _f32),
        pltpu.VMEM((_CEE, _D), _f32),
        pltpu.VMEM((_CEE, _D), _f32),
        pltpu.VMEM((_CEE, _D), _f32),
        pltpu.VMEM((_CEE, 128), _f32),
        pltpu.VMEM((_CEE, 128), _f32),
        pltpu.VMEM((_CEE, 128), _f32),
        pltpu.VMEM((_CEE, 128), _f32),
        pltpu.VMEM_SHARED((2 * _AGR, 128), _f32),
        pltpu.SemaphoreType.DMA,
        pltpu.SemaphoreType.DMA,
        pltpu.SemaphoreType.DMA,
        pltpu.SemaphoreType.DMA,
        pltpu.SemaphoreType.DMA,
        pltpu.SemaphoreType.DMA,
        pltpu.SemaphoreType.DMA,
    ],
)(_edge_agg_body)


def kernel(x, edge_index, edge_type, edge_weight, issuer_index, params):
    p = params
    src = jnp.pad(edge_index[0].astype(_i32), (0, _EP - _E))
    dst = jnp.pad(edge_index[1].astype(_i32), (0, _EP - _E),
                  constant_values=_N)
    typ = jnp.pad(edge_type.astype(_i32), (0, _EP - _E))
    w = jnp.pad(edge_weight.astype(_f32), (0, _EP - _E))
    ep2 = jnp.stack([src, dst], axis=1).reshape(_EP * 2 // 128, 128)
    ep4 = jnp.stack(
        [src, dst, typ, jax.lax.bitcast_convert_type(w, _i32)],
        axis=1).reshape(_EP * 4 // 128, 128)
    iss_idx = jnp.clip(issuer_index + 1, 0, 1000).astype(_i32)
    iss_idx = jnp.pad(iss_idx, (0, _NP - _N))

    sc = 1.0 / jnp.sqrt(1.0 + _EPS)
    s_in = p['g_in'] * sc
    c_in = p['b_in'] * s_in + p['beta_in']
    iss_tab = p['issuer_emb'] @ (p['W_in'][128:] * s_in[None, :])
    issrows = _iss_gather(iss_tab, iss_idx)

    xp = jnp.pad(x, ((0, _NP - _N), (0, 0)))
    h = _proj_in(xp, issrows, p['W_in'][:128] * s_in[None, :], c_in[None, :])

    g16 = jnp.pad(jnp.exp(p['rel_log_gain']), (0, 120)).astype(_f32)[None, :]
    z128 = jnp.zeros((_NP, 128), _f32)
    zagg = jnp.zeros((2 * _AGR, 128), _f32)

    for li in ('1', '2'):
        cp = p['conv' + li]
        r8 = p['rel_emb'] @ cp['We'][:16]
        u = cp['We'][16][None, :]
        attf = cp['att'].reshape(_D)[None, :]
        xl, xr = _proj_lr(h, cp['Wl'], cp['Wr'], cp['bl'][None, :],
                          cp['br'][None, :])
        ex = _edge_logits(xl, xr, ep4, r8, u, attf, g16)
        den01 = _den_acc(ex, ep2, z128)
        den4f = (den01[:_NP, :4] + den01[_NP:, :4]).reshape(_DFR, 128)
        af = _alpha(ex, dst, den4f)
        agg = _edge_agg(xl, ep2, af, zagg).reshape(_NP, _D)
        s_l = p['g' + li] * sc
        c_l = cp['bias'] * s_l + p['beta' + li]
        h = _affine_relu(agg, s_l[None, :], c_l[None, :])
    return h[:_N]


# speedup vs baseline: 1.0595x; 1.0595x over previous
import functools

import jax
import jax.numpy as jnp
from jax import lax
from jax.experimental import pallas as pl
from jax.experimental.pallas import tpu as pltpu
from jax.experimental.pallas import tpu_sc as plsc

_N = 10000
_E = 320000
_D = 256
_H = 4
_O = 64
_EPS = 1e-5
_NP = 10240
_BR = 512

_NW = 32
_EP = 327680
_CE = 32
_GC = 8
_EC = _EP // _NW
_NCHD = _EC // _CE
_CEA = 256
_CED = 64
_NCHA = (_EP // _NW) // _CEA
_CEE = 32
_ETE = _EP // 16
_NCHE = _ETE // _CEE
_DFR = _NP * 4 // 128
_HN = _NP // 2
_AGR = _HN + 128

_mesh = plsc.VectorSubcoreMesh(core_axis_name="c", subcore_axis_name="s")
_f32 = jnp.float32
_i32 = jnp.int32


def _proj_in_body(x_ref, iss_ref, wx_ref, c_ref, o_ref):
    acc = jnp.dot(x_ref[...], wx_ref[...], preferred_element_type=_f32)
    o_ref[...] = jnp.maximum(acc + iss_ref[...] + c_ref[...], 0.0)


def _proj_in(xp, issrows, wx, c):
    return pl.pallas_call(
        _proj_in_body,
        grid=(_NP // _BR,),
        in_specs=[
            pl.BlockSpec((_BR, 128), lambda i: (i, 0)),
            pl.BlockSpec((_BR, _D), lambda i: (i, 0)),
            pl.BlockSpec((128, _D), lambda i: (0, 0)),
            pl.BlockSpec((1, _D), lambda i: (0, 0)),
        ],
        out_specs=pl.BlockSpec((_BR, _D), lambda i: (i, 0)),
        out_shape=jax.ShapeDtypeStruct((_NP, _D), _f32),
    )(xp, issrows, wx, c)


def _lr_body(h_ref, wl_ref, wr_ref, bl_ref, br_ref, xl_ref, xr_ref):
    h = h_ref[...]
    xl_ref[...] = jnp.dot(h, wl_ref[...], preferred_element_type=_f32) + bl_ref[...]
    xr_ref[...] = jnp.dot(h, wr_ref[...], preferred_element_type=_f32) + br_ref[...]


def _proj_lr(h, wl, wr, bl, br):
    return pl.pallas_call(
        _lr_body,
        grid=(_NP // _BR,),
        in_specs=[
            pl.BlockSpec((_BR, _D), lambda i: (i, 0)),
            pl.BlockSpec((_D, _D), lambda i: (0, 0)),
            pl.BlockSpec((_D, _D), lambda i: (0, 0)),
            pl.BlockSpec((1, _D), lambda i: (0, 0)),
            pl.BlockSpec((1, _D), lambda i: (0, 0)),
        ],
        out_specs=[
            pl.BlockSpec((_BR, _D), lambda i: (i, 0)),
            pl.BlockSpec((_BR, _D), lambda i: (i, 0)),
        ],
        out_shape=[
            jax.ShapeDtypeStruct((_NP, _D), _f32),
            jax.ShapeDtypeStruct((_NP, _D), _f32),
        ],
    )(h, wl, wr, bl, br)


def _affine_relu_body(a_ref, s_ref, c_ref, o_ref):
    o_ref[...] = jnp.maximum(a_ref[...] * s_ref[...] + c_ref[...], 0.0)


def _affine_relu(agg, s, c):
    return pl.pallas_call(
        _affine_relu_body,
        grid=(_NP // _BR,),
        in_specs=[
            pl.BlockSpec((_BR, _D), lambda i: (i, 0)),
            pl.BlockSpec((1, _D), lambda i: (0, 0)),
            pl.BlockSpec((1, _D), lambda i: (0, 0)),
        ],
        out_specs=pl.BlockSpec((_BR, _D), lambda i: (i, 0)),
        out_shape=jax.ShapeDtypeStruct((_NP, _D), _f32),
    )(agg, s, c)


@functools.partial(
    pl.kernel,
    out_type=jax.ShapeDtypeStruct((_NP, _D), _f32),
    mesh=_mesh,
    compiler_params=pltpu.CompilerParams(needs_layout_passes=False),
    scratch_types=[
        pltpu.VMEM((_NP // _NW,), _i32),
        pltpu.VMEM((_NP // _NW, _D), _f32),
        pltpu.SemaphoreType.DMA,
    ],
)
def _iss_gather(emb_hbm, idx_hbm, out_hbm, idx_v, rows_v, sem):
    wid = lax.axis_index("s") * 2 + lax.axis_index("c")
    base = wid * (_NP // _NW)
    pltpu.sync_copy(idx_hbm.at[pl.ds(base, _NP // _NW)], idx_v)
    pltpu.async_copy(emb_hbm.at[idx_v], rows_v, sem).wait()
    pltpu.sync_copy(rows_v, out_hbm.at[pl.ds(base, _NP // _NW)])


def _edge_logits_body(xl_hbm, xr_hbm, ep4_hbm, r8_hbm, u_hbm, att_hbm,
                      g16_hbm, ex_hbm,
                      src0_v, src1_v, src2_v, dst0_v, dst1_v, dst2_v,
                      ep4f_v, exf_v, xl0_v, xl1_v, xl2_v, xr0_v, xr1_v, xr2_v,
                      r8_v, u_v, att_v, g16_v,
                      gl0_sem, gl1_sem, gl2_sem, gr0_sem, gr1_sem, gr2_sem):
    c = lax.axis_index("c")
    s = lax.axis_index("s")
    wid = s * 2 + c
    pltpu.sync_copy(r8_hbm, r8_v)
    pltpu.sync_copy(u_hbm, u_v)
    pltpu.sync_copy(att_hbm, att_v)
    pltpu.sync_copy(g16_hbm, g16_v)

    iota = lax.iota(_i32, 16)
    zid = jnp.zeros((16,), _i32)
    srcs = (src0_v, src1_v, src2_v)
    dsts = (dst0_v, dst1_v, dst2_v)
    xls = (xl0_v, xl1_v, xl2_v)
    xrs = (xr0_v, xr1_v, xr2_v)
    glsems = (gl0_sem, gl1_sem, gl2_sem)
    grsems = (gr0_sem, gr1_sem, gr2_sem)
    nchunk = _EC // _CE
    ngrp = nchunk // _GC
    erow4 = _EC * 4 // 128

    def extract(k):
        b = k % 3
        for g in range(_CE // 16):
            sl = pl.ds(g * 16, 16)
            f4 = (k * _CE + g * 16 + iota) * 4
            sv = plsc.load_gather(ep4f_v, [f4 >> 7, f4 & 127])
            dv = plsc.load_gather(ep4f_v, [f4 >> 7, (f4 & 127) + 1])
            srcs[b][sl] = sv
            dsts[b][sl] = dv

    def launch(k):
        b = k % 3
        return (pltpu.async_copy(xl_hbm.at[srcs[b]], xls[b], glsems[b]),
                pltpu.async_copy(xr_hbm.at[dsts[b]], xrs[b], grsems[b]))

    def grp(og, _):
        pltpu.sync_copy(
            ep4_hbm.at[pl.ds(wid * erow4 + og * (_CE * _GC * 4 // 128),
                             _CE * _GC * 4 // 128)], ep4f_v)
        pend_g = {}
        extract(0)
        pend_g[0] = launch(0)
        extract(1)
        pend_g[1] = launch(1)
        for k in range(_GC):
            b = k % 3
            if k < _GC - 2:
                extract(k + 2)
                pend_g[(k + 2) % 3] = launch(k + 2)
            pend_g[b][0].wait()
            pend_g[b][1].wait()
            tgs, wgs = [], []
            for g in range(_CE // 16):
                f4 = (k * _CE + g * 16 + iota) * 4
                tv = plsc.load_gather(ep4f_v, [f4 >> 7, (f4 & 127) + 2])
                wv = plsc.bitcast(
                    plsc.load_gather(ep4f_v, [f4 >> 7, (f4 & 127) + 3]), _f32)
                gg = plsc.load_gather(g16_v, [zid, tv])
                tgs.append(tv)
                wgs.append(wv * gg)
            for h in range(_H):

                @plsc.parallel_loop(0, _O, 1, unroll=4,
                                    carry=tuple(jnp.zeros((16,), _f32)
                                                for _ in range(_CE // 16)))
                def accs(j, acc, b=b, h=h, tgs=tgs, wgs=wgs):
                    colv = jnp.full((16,), h * _O, _i32) + j
                    uj = plsc.load_gather(u_v, [zid, colv])
                    aj = plsc.load_gather(att_v, [zid, colv])
                    out = []
                    for g in range(_CE // 16):
                        iog = iota + g * 16
                        xlg = plsc.load_gather(xls[b], [iog, colv])
                        xrg = plsc.load_gather(xrs[b], [iog, colv])
                        r8g = plsc.load_gather(r8_v, [tgs[g], colv])
                        m = xlg + xrg + r8g + wgs[g] * uj
                        m = jnp.where(m >= 0.0, m, m * 0.2)
                        out.append(acc[g] + aj * m)
                    return tuple(out)

                for g in range(_CE // 16):
                    exv = jnp.exp(accs[g])
                    f = (k * _CE + g * 16 + iota) * 4 + h
                    plsc.store_scatter(exf_v, [f >> 7, f & 127], exv)
        pltpu.sync_copy(
            exf_v,
            ex_hbm.at[pl.ds(wid * erow4 + og * (_CE * _GC * 4 // 128),
                            _CE * _GC * 4 // 128)])
        return 0

    lax.fori_loop(0, ngrp, grp, 0)


_edge_logits = functools.partial(
    pl.kernel,
    out_type=jax.ShapeDtypeStruct((_EP * 4 // 128, 128), _f32),
    mesh=_mesh,
    compiler_params=pltpu.CompilerParams(needs_layout_passes=False),
    scratch_types=[
        pltpu.VMEM((_CE,), _i32),
        pltpu.VMEM((_CE,), _i32),
        pltpu.VMEM((_CE,), _i32),
        pltpu.VMEM((_CE,), _i32),
        pltpu.VMEM((_CE,), _i32),
        pltpu.VMEM((_CE,), _i32),
        pltpu.VMEM((_CE * _GC * 4 // 128, 128), _i32),
        pltpu.VMEM((_CE * _GC * 4 // 128, 128), _f32),
        pltpu.VMEM((_CE, _D), _f32),
        pltpu.VMEM((_CE, _D), _f32),
        pltpu.VMEM((_CE, _D), _f32),
        pltpu.VMEM((_CE, _D), _f32),
        pltpu.VMEM((_CE, _D), _f32),
        pltpu.VMEM((_CE, _D), _f32),
        pltpu.VMEM((8, _D), _f32),
        pltpu.VMEM((1, _D), _f32),
        pltpu.VMEM((1, _D), _f32),
        pltpu.VMEM((1, 128), _f32),
        pltpu.SemaphoreType.DMA,
        pltpu.SemaphoreType.DMA,
        pltpu.SemaphoreType.DMA,
        pltpu.SemaphoreType.DMA,
        pltpu.SemaphoreType.DMA,
        pltpu.SemaphoreType.DMA,
    ],
)(_edge_logits_body)


def _den_body(ex_hbm, ep2_hbm, z128_hbm, den01_hbm,
              dst0_v, dst1_v, dst2_v, dst3_v, ep2f_v, exf_v,
              exa_v, exb_v, den_sh, sd0_sem, sd1_sem):
    c = lax.axis_index("c")
    s = lax.axis_index("s")
    zsl = pl.ds(s * (_NP // 16), _NP // 16)
    pltpu.sync_copy(z128_hbm.at[zsl], den_sh.at[zsl])
    pltpu.sync_copy(z128_hbm.at[pl.ds(0, _CED)], exa_v)
    pltpu.sync_copy(z128_hbm.at[pl.ds(0, _CED)], exb_v)
    plsc.subcore_barrier()
    wid = s * 2 + c

    iota = lax.iota(_i32, 16)
    dsts = (dst0_v, dst1_v, dst2_v, dst3_v)
    exs = (exa_v, exb_v)
    sdsems = (sd0_sem, sd1_sem)
    epw = _EC * 2 // 128
    exw = _EC * 4 // 128

    def grp(og, _):
        pltpu.sync_copy(
            ep2_hbm.at[pl.ds(wid * epw + og * (_CED * 8 * 2 // 128),
                             _CED * 8 * 2 // 128)], ep2f_v)
        pltpu.sync_copy(
            ex_hbm.at[pl.ds(wid * exw + og * (_CED * 8 * 4 // 128),
                            _CED * 8 * 4 // 128)], exf_v)
        pend = {0: None, 1: None}
        for k in range(8):
            b = k % 2
            b4 = k % 4
            for g in range(_CED // 16):
                sl = pl.ds(g * 16, 16)
                f2 = (k * _CED + g * 16 + iota) * 2
                dv = plsc.load_gather(ep2f_v, [f2 >> 7, (f2 & 127) + 1])
                dsts[b4][sl] = dv
            if pend[b] is not None:
                pend[b].wait()
            for v in range(_CED * 4 // 16):
                f = v * 16 + k * _CED * 4 + iota
                fl = v * 16 + iota
                exv = plsc.load_gather(exf_v, [f >> 7, f & 127])
                plsc.store_scatter(exs[b], [fl >> 2, fl & 3], exv)
            pend[b] = pltpu.async_copy(exs[b], den_sh.at[dsts[b4]],
                                       sdsems[b], add=True)
        for b in (0, 1):
            if pend[b] is not None:
                pend[b].wait()
        return 0

    lax.fori_loop(0, _EC // (_CED * 8), grp, 0)
    plsc.subcore_barrier()
    pltpu.sync_copy(den_sh.at[zsl],
                    den01_hbm.at[pl.ds(c * _NP + s * (_NP // 16), _NP // 16)])


_den_acc = functools.partial(
    pl.kernel,
    out_type=jax.ShapeDtypeStruct((2 * _NP, 128), _f32),
    mesh=_mesh,
    compiler_params=pltpu.CompilerParams(needs_layout_passes=False),
    scratch_types=[
        pltpu.VMEM((_CED,), _i32),
        pltpu.VMEM((_CED,), _i32),
        pltpu.VMEM((_CED,), _i32),
        pltpu.VMEM((_CED,), _i32),
        pltpu.VMEM((_CED * 8 * 2 // 128, 128), _i32),
        pltpu.VMEM((_CED * 8 * 4 // 128, 128), _f32),
        pltpu.VMEM((_CED, 128), _f32),
        pltpu.VMEM((_CED, 128), _f32),
        pltpu.VMEM_SHARED((_NP, 128), _f32),
        pltpu.SemaphoreType.DMA,
        pltpu.SemaphoreType.DMA,
    ],
)(_den_body)


def _alpha_body(ex_hbm, dst_hbm, den4f_hbm, af_hbm,
                exf_v, dst_v, den4f_v, af_v):
    c = lax.axis_index("c")
    s = lax.axis_index("s")
    wid = s * 2 + c
    pltpu.sync_copy(den4f_hbm, den4f_v)
    iota = lax.iota(_i32, 16)

    def chunk(ci, _):
        base = wid * (_EP // _NW) + ci * _CEA
        rowb = wid * (_EP // _NW * 4 // 128) + ci * (_CEA * 4 // 128)
        pltpu.sync_copy(dst_hbm.at[pl.ds(base, _CEA)], dst_v)
        pltpu.sync_copy(ex_hbm.at[pl.ds(rowb, _CEA * 4 // 128)], exf_v)
        for v in range(_CEA * 4 // 16):
            f = iota + v * 16
            e = f >> 2
            hh = f & 3
            exv = plsc.load_gather(exf_v, [f >> 7, f & 127])
            dg = plsc.load_gather(dst_v, [e])
            gf = dg * 4 + hh
            den = plsc.load_gather(den4f_v, [gf >> 7, gf & 127])
            plsc.store_scatter(af_v, [f >> 7, f & 127], exv / den)
        pltpu.sync_copy(af_v, af_hbm.at[pl.ds(rowb, _CEA * 4 // 128)])
        return 0

    lax.fori_loop(0, _NCHA, chunk, 0)


_alpha = functools.partial(
    pl.kernel,
    out_type=jax.ShapeDtypeStruct((_EP * 4 // 128, 128), _f32),
    mesh=_mesh,
    compiler_params=pltpu.CompilerParams(needs_layout_passes=False),
    scratch_types=[
        pltpu.VMEM((_CEA * 4 // 128, 128), _f32),
        pltpu.VMEM((_CEA,), _i32),
        pltpu.VMEM((_DFR, 128), _f32),
        pltpu.VMEM((_CEA * 4 // 128, 128), _f32),
    ],
)(_alpha_body)


def _edge_agg_body(xl_hbm, ep2_hbm, af_hbm, zagg_hbm, agg_hbm,
                   src0_v, src1_v, src2_v, la0_v, la1_v, la2_v, la3_v,
                   lb0_v, lb1_v, lb2_v, lb3_v,
                   epf_v, af_v, xl0_v, xl1_v, xl2_v,
                   ma0_v, mb0_v, ma1_v, mb1_v, agg_sh,
                   g0_sem, g1_sem, g2_sem,
                   s0a_sem, s0b_sem, s1a_sem, s1b_sem):
    c = lax.axis_index("c")
    s = lax.axis_index("s")
    lo = c * _HN
    zsl = pl.ds(s * (2 * _AGR // 16), 2 * _AGR // 16)
    pltpu.sync_copy(zagg_hbm.at[zsl], agg_sh.at[zsl])
    plsc.subcore_barrier()

    iota = lax.iota(_i32, 16)
    srcs = (src0_v, src1_v, src2_v)
    las = (la0_v, la1_v, la2_v, la3_v)
    lbs = (lb0_v, lb1_v, lb2_v, lb3_v)
    xls = (xl0_v, xl1_v, xl2_v)
    mas = (ma0_v, ma1_v)
    mbs = (mb0_v, mb1_v)
    gsems = (g0_sem, g1_sem, g2_sem)
    sasems = (s0a_sem, s0b_sem)
    sbsems = (s1a_sem, s1b_sem)
    ngrp = _NCHE // 16
    erow2 = _ETE * 2 // 128
    erow4 = _ETE * 4 // 128

    def extract(k):
        b = k % 3
        b4 = k % 4
        for g in range(_CEE // 16):
            sl = pl.ds(g * 16, 16)
            le = k * _CEE + g * 16 + iota
            f2 = le * 2
            sv = plsc.load_gather(epf_v, [f2 >> 7, f2 & 127])
            dv = plsc.load_gather(epf_v, [f2 >> 7, (f2 & 127) + 1])
            srcs[b][sl] = sv
            inr = (dv >= lo) & (dv < lo + _HN)
            loc2 = jnp.where(inr, dv - lo, _HN) * 2
            las[b4][sl] = loc2
            lbs[b4][sl] = loc2 + 1

    def launch(k):
        b = k % 3
        return pltpu.async_copy(xl_hbm.at[srcs[b]], xls[b], gsems[b])

    def grp(og, _):
        pltpu.sync_copy(
            ep2_hbm.at[pl.ds(s * erow2 + og * 8, 8)], epf_v)
        pltpu.sync_copy(
            af_hbm.at[pl.ds(s * erow4 + og * 16, 16)], af_v)
        pend_g = {}
        pend_sa = {0: None, 1: None}
        extract(0)
        pend_g[0] = launch(0)
        extract(1)
        pend_g[1] = launch(1)
        for k in range(16):
            b = k % 2
            g3 = k % 3
            if k < 14:
                extract(k + 2)
                pend_g[(k + 2) % 3] = launch(k + 2)
            pend_g[g3].wait()
            alphas = []
            for h in range(_H):
                row = []
                for g in range(_CEE // 16):
                    f = (k * _CEE + g * 16 + iota) * 4 + h
                    row.append(plsc.load_gather(af_v, [f >> 7, f & 127]))
                alphas.append(row)
            if pend_sa[b] is not None:
                pend_sa[b][0].wait()
                pend_sa[b][1].wait()

            @plsc.parallel_loop(0, _O, 1, unroll=8)
            def _(j, b=b, g3=g3, alphas=alphas):
                mcolv = jnp.full((16,), j, _i32)
                mcolv2 = mcolv + _O
                for h in range(_H):
                    mref = mas[b] if h < 2 else mbs[b]
                    mc = mcolv if h % 2 == 0 else mcolv2
                    colv = mcolv + h * _O
                    for g in range(_CEE // 16):
                        iog = iota + g * 16
                        xlg = plsc.load_gather(xls[g3], [iog, colv])
                        plsc.store_scatter(mref, [iog, mc],
                                           alphas[h][g] * xlg)
            da = pltpu.async_copy(mas[b], agg_sh.at[las[k % 4]], sasems[b],
                                  add=True)
            db = pltpu.async_copy(mbs[b], agg_sh.at[lbs[k % 4]], sbsems[b],
                                  add=True)
            pend_sa[b] = (da, db)
        for b in (0, 1):
            if pend_sa[b] is not None:
                pend_sa[b][0].wait()
                pend_sa[b][1].wait()
        return 0

    lax.fori_loop(0, ngrp, grp, 0)
    plsc.subcore_barrier()
    osl = pl.ds(s * (2 * _HN // 16), 2 * _HN // 16)
    pltpu.sync_copy(
        agg_sh.at[osl],
        agg_hbm.at[pl.ds(2 * lo + s * (2 * _HN // 16), 2 * _HN // 16)])


_edge_agg = functools.partial(
    pl.kernel,
    out_type=jax.ShapeDtypeStruct((2 * _NP, 128), _f32),
    mesh=_mesh,
    compiler_params=pltpu.CompilerParams(needs_layout_passes=False),
    scratch_types=[
        pltpu.VMEM((_CEE,), _i32),
        pltpu.VMEM((_CEE,), _i32),
        pltpu.VMEM((_CEE,), _i32),
        pltpu.VMEM((_CEE,), _i32),
        pltpu.VMEM((_CEE,), _i32),
        pltpu.VMEM((_CEE,), _i32),
        pltpu.VMEM((_CEE,), _i32),
        pltpu.VMEM((_CEE,), _i32),
        pltpu.VMEM((_CEE,), _i32),
        pltpu.VMEM((_CEE,), _i32),
        pltpu.VMEM((_CEE,), _i32),
        pltpu.VMEM((8, 128), _i32),
        pltpu.VMEM((16, 128), _f32),
        pltpu.VMEM((_CEE, _D), _f32),
        pltpu.VMEM((_CEE, _D), _f32),
        pltpu.VMEM((_CEE, _D), _f32),
        pltpu.VMEM((_CEE, 128), _f32),
        pltpu.VMEM((_CEE, 128), _f32),
        pltpu.VMEM((_CEE, 128), _f32),
        pltpu.VMEM((_CEE, 128), _f32),
        pltpu.VMEM_SHARED((2 * _AGR, 128), _f32),
        pltpu.SemaphoreType.DMA,
        pltpu.SemaphoreType.DMA,
        pltpu.SemaphoreType.DMA,
        pltpu.SemaphoreType.DMA,
        pltpu.SemaphoreType.DMA,
        pltpu.SemaphoreType.DMA,
        pltpu.SemaphoreType.DMA,
    ],
)(_edge_agg_body)


def kernel(x, edge_index, edge_type, edge_weight, issuer_index, params):
    p = params
    src = jnp.pad(edge_index[0].astype(_i32), (0, _EP - _E))
    dst = jnp.pad(edge_index[1].astype(_i32), (0, _EP - _E),
                  constant_values=_N)
    typ = jnp.pad(edge_type.astype(_i32), (0, _EP - _E))
    w = jnp.pad(edge_weight.astype(_f32), (0, _EP - _E))
    ep2 = jnp.stack([src, dst], axis=1).reshape(_EP * 2 // 128, 128)
    ep4 = jnp.stack(
        [src, dst, typ, jax.lax.bitcast_convert_type(w, _i32)],
        axis=1).reshape(_EP * 4 // 128, 128)
    iss_idx = jnp.clip(issuer_index + 1, 0, 1000).astype(_i32)
    iss_idx = jnp.pad(iss_idx, (0, _NP - _N))

    sc = 1.0 / jnp.sqrt(1.0 + _EPS)
    s_in = p['g_in'] * sc
    c_in = p['b_in'] * s_in + p['beta_in']
    iss_tab = p['issuer_emb'] @ (p['W_in'][128:] * s_in[None, :])
    issrows = _iss_gather(iss_tab, iss_idx)

    xp = jnp.pad(x, ((0, _NP - _N), (0, 0)))
    h = _proj_in(xp, issrows, p['W_in'][:128] * s_in[None, :], c_in[None, :])

    g16 = jnp.pad(jnp.exp(p['rel_log_gain']), (0, 120)).astype(_f32)[None, :]
    z128 = jnp.zeros((_NP, 128), _f32)
    zagg = jnp.zeros((2 * _AGR, 128), _f32)

    for li in ('1', '2'):
        cp = p['conv' + li]
        r8 = p['rel_emb'] @ cp['We'][:16]
        u = cp['We'][16][None, :]
        attf = cp['att'].reshape(_D)[None, :]
        xl, xr = _proj_lr(h, cp['Wl'], cp['Wr'], cp['bl'][None, :],
                          cp['br'][None, :])
        ex = _edge_logits(xl, xr, ep4, r8, u, attf, g16)
        den01 = _den_acc(ex, ep2, z128)
        den4f = (den01[:_NP, :4] + den01[_NP:, :4]).reshape(_DFR, 128)
        af = _alpha(ex, dst, den4f)
        agg = _edge_agg(xl, ep2, af, zagg).reshape(_NP, _D)
        s_l = p['g' + li] * sc
        c_l = cp['bias'] * s_l + p['beta' + li]
        h = _affine_relu(agg, s_l[None, :], c_l[None, :])
    return h[:_N]


# speedup vs baseline: 1.0631x; 1.0034x over previous
import functools

import jax
import jax.numpy as jnp
from jax import lax
from jax.experimental import pallas as pl
from jax.experimental.pallas import tpu as pltpu
from jax.experimental.pallas import tpu_sc as plsc

_N = 10000
_E = 320000
_D = 256
_H = 4
_O = 64
_EPS = 1e-5
_NP = 10240
_BR = 512

_NW = 32
_EP = 327680
_CE = 32
_EC = _EP // _NW
_NCHD = _EC // _CE
_CEA = 256
_NCHA = (_EP // _NW) // _CEA
_CEE = 32
_ETE = _EP // 16
_NCHE = _ETE // _CEE
_DFR = _NP * 4 // 128
_HN = _NP // 2
_AGR = _HN + 128

_mesh = plsc.VectorSubcoreMesh(core_axis_name="c", subcore_axis_name="s")
_f32 = jnp.float32
_i32 = jnp.int32


def _proj_in_body(x_ref, iss_ref, wx_ref, c_ref, o_ref):
    acc = jnp.dot(x_ref[...], wx_ref[...], preferred_element_type=_f32)
    o_ref[...] = jnp.maximum(acc + iss_ref[...] + c_ref[...], 0.0)


def _proj_in(xp, issrows, wx, c):
    return pl.pallas_call(
        _proj_in_body,
        grid=(_NP // _BR,),
        in_specs=[
            pl.BlockSpec((_BR, 128), lambda i: (i, 0)),
            pl.BlockSpec((_BR, _D), lambda i: (i, 0)),
            pl.BlockSpec((128, _D), lambda i: (0, 0)),
            pl.BlockSpec((1, _D), lambda i: (0, 0)),
        ],
        out_specs=pl.BlockSpec((_BR, _D), lambda i: (i, 0)),
        out_shape=jax.ShapeDtypeStruct((_NP, _D), _f32),
    )(xp, issrows, wx, c)


def _lr_body(h_ref, wl_ref, wr_ref, bl_ref, br_ref, xl_ref, xr_ref):
    h = h_ref[...]
    xl_ref[...] = jnp.dot(h, wl_ref[...], preferred_element_type=_f32) + bl_ref[...]
    xr_ref[...] = jnp.dot(h, wr_ref[...], preferred_element_type=_f32) + br_ref[...]


def _proj_lr(h, wl, wr, bl, br):
    return pl.pallas_call(
        _lr_body,
        grid=(_NP // _BR,),
        in_specs=[
            pl.BlockSpec((_BR, _D), lambda i: (i, 0)),
            pl.BlockSpec((_D, _D), lambda i: (0, 0)),
            pl.BlockSpec((_D, _D), lambda i: (0, 0)),
            pl.BlockSpec((1, _D), lambda i: (0, 0)),
            pl.BlockSpec((1, _D), lambda i: (0, 0)),
        ],
        out_specs=[
            pl.BlockSpec((_BR, _D), lambda i: (i, 0)),
            pl.BlockSpec((_BR, _D), lambda i: (i, 0)),
        ],
        out_shape=[
            jax.ShapeDtypeStruct((_NP, _D), _f32),
            jax.ShapeDtypeStruct((_NP, _D), _f32),
        ],
    )(h, wl, wr, bl, br)


def _affine_relu_body(a_ref, s_ref, c_ref, o_ref):
    o_ref[...] = jnp.maximum(a_ref[...] * s_ref[...] + c_ref[...], 0.0)


def _affine_relu(agg, s, c):
    return pl.pallas_call(
        _affine_relu_body,
        grid=(_NP // _BR,),
        in_specs=[
            pl.BlockSpec((_BR, _D), lambda i: (i, 0)),
            pl.BlockSpec((1, _D), lambda i: (0, 0)),
            pl.BlockSpec((1, _D), lambda i: (0, 0)),
        ],
        out_specs=pl.BlockSpec((_BR, _D), lambda i: (i, 0)),
        out_shape=jax.ShapeDtypeStruct((_NP, _D), _f32),
    )(agg, s, c)


@functools.partial(
    pl.kernel,
    out_type=jax.ShapeDtypeStruct((_NP, _D), _f32),
    mesh=_mesh,
    compiler_params=pltpu.CompilerParams(needs_layout_passes=False),
    scratch_types=[
        pltpu.VMEM((_NP // _NW,), _i32),
        pltpu.VMEM((_NP // _NW, _D), _f32),
        pltpu.SemaphoreType.DMA,
    ],
)
def _iss_gather(emb_hbm, idx_hbm, out_hbm, idx_v, rows_v, sem):
    wid = lax.axis_index("s") * 2 + lax.axis_index("c")
    base = wid * (_NP // _NW)
    pltpu.sync_copy(idx_hbm.at[pl.ds(base, _NP // _NW)], idx_v)
    pltpu.async_copy(emb_hbm.at[idx_v], rows_v, sem).wait()
    pltpu.sync_copy(rows_v, out_hbm.at[pl.ds(base, _NP // _NW)])


def _edge_logits_body(xl_hbm, xr_hbm, ep4_hbm, r8_hbm, u_hbm, att_hbm,
                      g16_hbm, z128_hbm, ex_hbm, den01_hbm,
                      src0_v, src1_v, dst0_v, dst1_v, dst2_v, dst3_v,
                      ep4f_v, exf_v, xl0_v, xl1_v, xr0_v, xr1_v,
                      exa_v, exb_v, r8_v, u_v, att_v, g16_v, den_sh,
                      gl0_sem, gl1_sem, gr0_sem, gr1_sem, sd0_sem, sd1_sem):
    c = lax.axis_index("c")
    s = lax.axis_index("s")
    wid = s * 2 + c
    pltpu.sync_copy(r8_hbm, r8_v)
    pltpu.sync_copy(u_hbm, u_v)
    pltpu.sync_copy(att_hbm, att_v)
    pltpu.sync_copy(g16_hbm, g16_v)
    zsl = pl.ds(s * (_NP // 16), _NP // 16)
    pltpu.sync_copy(z128_hbm.at[zsl], den_sh.at[zsl])
    pltpu.sync_copy(z128_hbm.at[pl.ds(0, _CE)], exa_v)
    pltpu.sync_copy(z128_hbm.at[pl.ds(0, _CE)], exb_v)
    plsc.subcore_barrier()

    iota = lax.iota(_i32, 16)
    zid = jnp.zeros((16,), _i32)
    srcs = (src0_v, src1_v)
    dsts = (dst0_v, dst1_v, dst2_v, dst3_v)
    xls = (xl0_v, xl1_v)
    xrs = (xr0_v, xr1_v)
    exs = (exa_v, exb_v)
    glsems = (gl0_sem, gl1_sem)
    grsems = (gr0_sem, gr1_sem)
    sdsems = (sd0_sem, sd1_sem)
    ngrp = _NCHD // 8
    erow4 = _EC * 4 // 128

    def extract(k):
        b = k % 2
        b4 = k % 4
        for g in range(_CE // 16):
            sl = pl.ds(g * 16, 16)
            f4 = (k * _CE + g * 16 + iota) * 4
            sv = plsc.load_gather(ep4f_v, [f4 >> 7, f4 & 127])
            dv = plsc.load_gather(ep4f_v, [f4 >> 7, (f4 & 127) + 1])
            srcs[b][sl] = sv
            dsts[b4][sl] = dv

    def launch(k):
        b = k % 2
        return (pltpu.async_copy(xl_hbm.at[srcs[b]], xls[b], glsems[b]),
                pltpu.async_copy(xr_hbm.at[dsts[k % 4]], xrs[b], grsems[b]))

    def grp(og, _):
        pltpu.sync_copy(
            ep4_hbm.at[pl.ds(wid * erow4 + og * 8, 8)], ep4f_v)
        pend_g = {}
        pend_sd = {0: None, 1: None}
        extract(0)
        pend_g[0] = launch(0)
        for k in range(8):
            b = k % 2
            if k < 7:
                extract(k + 1)
                pend_g[(k + 1) % 2] = launch(k + 1)
            pend_g[b][0].wait()
            pend_g[b][1].wait()
            tgs, wgs = [], []
            for g in range(_CE // 16):
                f4 = (k * _CE + g * 16 + iota) * 4
                tv = plsc.load_gather(ep4f_v, [f4 >> 7, (f4 & 127) + 2])
                wv = plsc.bitcast(
                    plsc.load_gather(ep4f_v, [f4 >> 7, (f4 & 127) + 3]), _f32)
                gg = plsc.load_gather(g16_v, [zid, tv])
                tgs.append(tv)
                wgs.append(wv * gg)
            if pend_sd[b] is not None:
                pend_sd[b].wait()
            for h in range(_H):
                hsplat = jnp.full((16,), h, _i32)

                @plsc.parallel_loop(0, _O, 1, unroll=4,
                                    carry=tuple(jnp.zeros((16,), _f32)
                                                for _ in range(_CE // 16)))
                def accs(j, acc, b=b, h=h, tgs=tgs, wgs=wgs):
                    colv = jnp.full((16,), h * _O, _i32) + j
                    uj = plsc.load_gather(u_v, [zid, colv])
                    aj = plsc.load_gather(att_v, [zid, colv])
                    out = []
                    for g in range(_CE // 16):
                        iog = iota + g * 16
                        xlg = plsc.load_gather(xls[b], [iog, colv])
                        xrg = plsc.load_gather(xrs[b], [iog, colv])
                        r8g = plsc.load_gather(r8_v, [tgs[g], colv])
                        m = xlg + xrg + r8g + wgs[g] * uj
                        m = jnp.where(m >= 0.0, m, m * 0.2)
                        out.append(acc[g] + aj * m)
                    return tuple(out)

                for g in range(_CE // 16):
                    iog = iota + g * 16
                    exv = jnp.exp(accs[g])
                    plsc.store_scatter(exs[b], [iog, hsplat], exv)
                    fcol = (g * 16 + iota) * 4 + h
                    plsc.store_scatter(exf_v,
                                       [jnp.full((16,), k, _i32), fcol], exv)
            pend_sd[b] = pltpu.async_copy(exs[b], den_sh.at[dsts[k % 4]],
                                          sdsems[b], add=True)
        for b in (0, 1):
            if pend_sd[b] is not None:
                pend_sd[b].wait()
        pltpu.sync_copy(exf_v,
                        ex_hbm.at[pl.ds(wid * erow4 + og * 8, 8)])
        return 0

    lax.fori_loop(0, ngrp, grp, 0)
    plsc.subcore_barrier()
    pltpu.sync_copy(den_sh.at[zsl],
                    den01_hbm.at[pl.ds(c * _NP + s * (_NP // 16), _NP // 16)])


_edge_logits = functools.partial(
    pl.kernel,
    out_type=[
        jax.ShapeDtypeStruct((_EP * 4 // 128, 128), _f32),
        jax.ShapeDtypeStruct((2 * _NP, 128), _f32),
    ],
    mesh=_mesh,
    compiler_params=pltpu.CompilerParams(needs_layout_passes=False),
    scratch_types=[
        pltpu.VMEM((_CE,), _i32),
        pltpu.VMEM((_CE,), _i32),
        pltpu.VMEM((_CE,), _i32),
        pltpu.VMEM((_CE,), _i32),
        pltpu.VMEM((_CE,), _i32),
        pltpu.VMEM((_CE,), _i32),
        pltpu.VMEM((8, 128), _i32),
        pltpu.VMEM((8, 128), _f32),
        pltpu.VMEM((_CE, _D), _f32),
        pltpu.VMEM((_CE, _D), _f32),
        pltpu.VMEM((_CE, _D), _f32),
        pltpu.VMEM((_CE, _D), _f32),
        pltpu.VMEM((_CE, 128), _f32),
        pltpu.VMEM((_CE, 128), _f32),
        pltpu.VMEM((8, _D), _f32),
        pltpu.VMEM((1, _D), _f32),
        pltpu.VMEM((1, _D), _f32),
        pltpu.VMEM((1, 128), _f32),
        pltpu.VMEM_SHARED((_NP, 128), _f32),
        pltpu.SemaphoreType.DMA,
        pltpu.SemaphoreType.DMA,
        pltpu.SemaphoreType.DMA,
        pltpu.SemaphoreType.DMA,
        pltpu.SemaphoreType.DMA,
        pltpu.SemaphoreType.DMA,
    ],
)(_edge_logits_body)


def _alpha_body(ex_hbm, dst_hbm, den4f_hbm, af_hbm,
                exf_v, dst_v, den4f_v, af_v):
    c = lax.axis_index("c")
    s = lax.axis_index("s")
    wid = s * 2 + c
    pltpu.sync_copy(den4f_hbm, den4f_v)
    iota = lax.iota(_i32, 16)

    def chunk(ci, _):
        base = wid * (_EP // _NW) + ci * _CEA
        rowb = wid * (_EP // _NW * 4 // 128) + ci * (_CEA * 4 // 128)
        pltpu.sync_copy(dst_hbm.at[pl.ds(base, _CEA)], dst_v)
        pltpu.sync_copy(ex_hbm.at[pl.ds(rowb, _CEA * 4 // 128)], exf_v)
        for v in range(_CEA * 4 // 16):
            f = iota + v * 16
            e = f >> 2
            hh = f & 3
            exv = plsc.load_gather(exf_v, [f >> 7, f & 127])
            dg = plsc.load_gather(dst_v, [e])
            gf = dg * 4 + hh
            den = plsc.load_gather(den4f_v, [gf >> 7, gf & 127])
            plsc.store_scatter(af_v, [f >> 7, f & 127], exv / den)
        pltpu.sync_copy(af_v, af_hbm.at[pl.ds(rowb, _CEA * 4 // 128)])
        return 0

    lax.fori_loop(0, _NCHA, chunk, 0)


_alpha = functools.partial(
    pl.kernel,
    out_type=jax.ShapeDtypeStruct((_EP * 4 // 128, 128), _f32),
    mesh=_mesh,
    compiler_params=pltpu.CompilerParams(needs_layout_passes=False),
    scratch_types=[
        pltpu.VMEM((_CEA * 4 // 128, 128), _f32),
        pltpu.VMEM((_CEA,), _i32),
        pltpu.VMEM((_DFR, 128), _f32),
        pltpu.VMEM((_CEA * 4 // 128, 128), _f32),
    ],
)(_alpha_body)


def _edge_agg_body(xl_hbm, ep2_hbm, af_hbm, zagg_hbm, agg_hbm,
                   src0_v, src1_v, src2_v, la0_v, la1_v, la2_v, la3_v,
                   lb0_v, lb1_v, lb2_v, lb3_v,
                   epf_v, af_v, xl0_v, xl1_v, xl2_v,
                   ma0_v, mb0_v, ma1_v, mb1_v, agg_sh,
                   g0_sem, g1_sem, g2_sem,
                   s0a_sem, s0b_sem, s1a_sem, s1b_sem):
    c = lax.axis_index("c")
    s = lax.axis_index("s")
    lo = c * _HN
    zsl = pl.ds(s * (2 * _AGR // 16), 2 * _AGR // 16)
    pltpu.sync_copy(zagg_hbm.at[zsl], agg_sh.at[zsl])
    plsc.subcore_barrier()

    iota = lax.iota(_i32, 16)
    srcs = (src0_v, src1_v, src2_v)
    las = (la0_v, la1_v, la2_v, la3_v)
    lbs = (lb0_v, lb1_v, lb2_v, lb3_v)
    xls = (xl0_v, xl1_v, xl2_v)
    mas = (ma0_v, ma1_v)
    mbs = (mb0_v, mb1_v)
    gsems = (g0_sem, g1_sem, g2_sem)
    sasems = (s0a_sem, s0b_sem)
    sbsems = (s1a_sem, s1b_sem)
    ngrp = _NCHE // 16
    erow2 = _ETE * 2 // 128
    erow4 = _ETE * 4 // 128

    def extract(k):
        b = k % 3
        b4 = k % 4
        for g in range(_CEE // 16):
            sl = pl.ds(g * 16, 16)
            le = k * _CEE + g * 16 + iota
            f2 = le * 2
            sv = plsc.load_gather(epf_v, [f2 >> 7, f2 & 127])
            dv = plsc.load_gather(epf_v, [f2 >> 7, (f2 & 127) + 1])
            srcs[b][sl] = sv
            inr = (dv >= lo) & (dv < lo + _HN)
            loc2 = jnp.where(inr, dv - lo, _HN) * 2
            las[b4][sl] = loc2
            lbs[b4][sl] = loc2 + 1

    def launch(k):
        b = k % 3
        return pltpu.async_copy(xl_hbm.at[srcs[b]], xls[b], gsems[b])

    def grp(og, _):
        pltpu.sync_copy(
            ep2_hbm.at[pl.ds(s * erow2 + og * 8, 8)], epf_v)
        pltpu.sync_copy(
            af_hbm.at[pl.ds(s * erow4 + og * 16, 16)], af_v)
        pend_g = {}
        pend_sa = {0: None, 1: None}
        extract(0)
        pend_g[0] = launch(0)
        extract(1)
        pend_g[1] = launch(1)
        for k in range(16):
            b = k % 2
            g3 = k % 3
            if k < 14:
                extract(k + 2)
                pend_g[(k + 2) % 3] = launch(k + 2)
            pend_g[g3].wait()
            alphas = []
            for h in range(_H):
                row = []
                for g in range(_CEE // 16):
                    f = (k * _CEE + g * 16 + iota) * 4 + h
                    row.append(plsc.load_gather(af_v, [f >> 7, f & 127]))
                alphas.append(row)
            if pend_sa[b] is not None:
                pend_sa[b][0].wait()
                pend_sa[b][1].wait()

            @plsc.parallel_loop(0, _O, 1, unroll=8)
            def _(j, b=b, g3=g3, alphas=alphas):
                mcolv = jnp.full((16,), j, _i32)
                mcolv2 = mcolv + _O
                for h in range(_H):
                    mref = mas[b] if h < 2 else mbs[b]
                    mc = mcolv if h % 2 == 0 else mcolv2
                    colv = mcolv + h * _O
                    for g in range(_CEE // 16):
                        iog = iota + g * 16
                        xlg = plsc.load_gather(xls[g3], [iog, colv])
                        plsc.store_scatter(mref, [iog, mc],
                                           alphas[h][g] * xlg)
            da = pltpu.async_copy(mas[b], agg_sh.at[las[k % 4]], sasems[b],
                                  add=True)
            db = pltpu.async_copy(mbs[b], agg_sh.at[lbs[k % 4]], sbsems[b],
                                  add=True)
            pend_sa[b] = (da, db)
        for b in (0, 1):
            if pend_sa[b] is not None:
                pend_sa[b][0].wait()
                pend_sa[b][1].wait()
        return 0

    lax.fori_loop(0, ngrp, grp, 0)
    plsc.subcore_barrier()
    osl = pl.ds(s * (2 * _HN // 16), 2 * _HN // 16)
    pltpu.sync_copy(
        agg_sh.at[osl],
        agg_hbm.at[pl.ds(2 * lo + s * (2 * _HN // 16), 2 * _HN // 16)])


_edge_agg = functools.partial(
    pl.kernel,
    out_type=jax.ShapeDtypeStruct((2 * _NP, 128), _f32),
    mesh=_mesh,
    compiler_params=pltpu.CompilerParams(needs_layout_passes=False),
    scratch_types=[
        pltpu.VMEM((_CEE,), _i32),
        pltpu.VMEM((_CEE,), _i32),
        pltpu.VMEM((_CEE,), _i32),
        pltpu.VMEM((_CEE,), _i32),
        pltpu.VMEM((_CEE,), _i32),
        pltpu.VMEM((_CEE,), _i32),
        pltpu.VMEM((_CEE,), _i32),
        pltpu.VMEM((_CEE,), _i32),
        pltpu.VMEM((_CEE,), _i32),
        pltpu.VMEM((_CEE,), _i32),
        pltpu.VMEM((_CEE,), _i32),
        pltpu.VMEM((8, 128), _i32),
        pltpu.VMEM((16, 128), _f32),
        pltpu.VMEM((_CEE, _D), _f32),
        pltpu.VMEM((_CEE, _D), _f32),
        pltpu.VMEM((_CEE, _D), _f32),
        pltpu.VMEM((_CEE, 128), _f32),
        pltpu.VMEM((_CEE, 128), _f32),
        pltpu.VMEM((_CEE, 128), _f32),
        pltpu.VMEM((_CEE, 128), _f32),
        pltpu.VMEM_SHARED((2 * _AGR, 128), _f32),
        pltpu.SemaphoreType.DMA,
        pltpu.SemaphoreType.DMA,
        pltpu.SemaphoreType.DMA,
        pltpu.SemaphoreType.DMA,
        pltpu.SemaphoreType.DMA,
        pltpu.SemaphoreType.DMA,
        pltpu.SemaphoreType.DMA,
    ],
)(_edge_agg_body)


def kernel(x, edge_index, edge_type, edge_weight, issuer_index, params):
    p = params
    src = jnp.pad(edge_index[0].astype(_i32), (0, _EP - _E))
    dst = jnp.pad(edge_index[1].astype(_i32), (0, _EP - _E),
                  constant_values=_N)
    typ = jnp.pad(edge_type.astype(_i32), (0, _EP - _E))
    w = jnp.pad(edge_weight.astype(_f32), (0, _EP - _E))
    ep2 = jnp.stack([src, dst], axis=1).reshape(_EP * 2 // 128, 128)
    ep4 = jnp.stack(
        [src, dst, typ, jax.lax.bitcast_convert_type(w, _i32)],
        axis=1).reshape(_EP * 4 // 128, 128)
    iss_idx = jnp.clip(issuer_index + 1, 0, 1000).astype(_i32)
    iss_idx = jnp.pad(iss_idx, (0, _NP - _N))

    sc = 1.0 / jnp.sqrt(1.0 + _EPS)
    s_in = p['g_in'] * sc
    c_in = p['b_in'] * s_in + p['beta_in']
    iss_tab = p['issuer_emb'] @ (p['W_in'][128:] * s_in[None, :])
    issrows = _iss_gather(iss_tab, iss_idx)

    xp = jnp.pad(x, ((0, _NP - _N), (0, 0)))
    h = _proj_in(xp, issrows, p['W_in'][:128] * s_in[None, :], c_in[None, :])

    g16 = jnp.pad(jnp.exp(p['rel_log_gain']), (0, 120)).astype(_f32)[None, :]
    z128 = jnp.zeros((_NP, 128), _f32)
    zagg = jnp.zeros((2 * _AGR, 128), _f32)

    for li in ('1', '2'):
        cp = p['conv' + li]
        r8 = p['rel_emb'] @ cp['We'][:16]
        u = cp['We'][16][None, :]
        attf = cp['att'].reshape(_D)[None, :]
        xl, xr = _proj_lr(h, cp['Wl'], cp['Wr'], cp['bl'][None, :],
                          cp['br'][None, :])
        ex, den01 = _edge_logits(xl, xr, ep4, r8, u, attf, g16, z128)
        den4f = (den01[:_NP, :4] + den01[_NP:, :4]).reshape(_DFR, 128)
        af = _alpha(ex, dst, den4f)
        agg = _edge_agg(xl, ep2, af, zagg).reshape(_NP, _D)
        s_l = p['g' + li] * sc
        c_l = cp['bias'] * s_l + p['beta' + li]
        h = _affine_relu(agg, s_l[None, :], c_l[None, :])
    return h[:_N]
